# baseline (speedup 1.0000x reference)
import jax
import jax.numpy as jnp
from jax.experimental import pallas as pl
from jax.experimental.pallas import tpu as pltpu

B = 16384
EMB = 32
VSIZES = (52, 14, 12, 14, 8)
PV = (56, 16, 16, 16, 8)
POFF = (0, 56, 72, 88, 104)
VPAD = 128
BLK = 2048


def _body(si_ref, gi_ref, oi_ref, ui_ref, mi_ref,
          st_ref, gt_ref, ot_ref, ut_ref, mt_ref,
          w1_ref, b1_ref, w2_ref, b2_ref, w3_ref, b3_ref,
          out_ref, m_ref):
    @pl.when(pl.program_id(0) == 0)
    def _fold():
        m_ref[...] = jnp.zeros((VPAD, 256), jnp.float32)
        for f, t_ref in enumerate((st_ref, gt_ref, ot_ref, ut_ref, mt_ref)):
            t = t_ref[...]
            if PV[f] > VSIZES[f]:
                t = jnp.concatenate(
                    [t, jnp.zeros((PV[f] - VSIZES[f], EMB), jnp.float32)], 0)
            w1f = w1_ref[f * EMB:(f + 1) * EMB, :]
            p = jnp.dot(t, w1f, preferred_element_type=jnp.float32)
            if f == 4:
                p = p + b1_ref[...]
            m_ref[POFF[f]:POFF[f] + PV[f], :] = p

    pieces = []
    for f, i_ref in enumerate((si_ref, gi_ref, oi_ref, ui_ref, mi_ref)):
        iota = jax.lax.broadcasted_iota(jnp.int32, (PV[f], BLK), 0)
        pieces.append((iota == i_ref[0]).astype(jnp.float32))
    pieces.append(jnp.zeros((VPAD - sum(PV), BLK), jnp.float32))
    a_t = jnp.concatenate(pieces, axis=0)

    h1 = jnp.maximum(
        jax.lax.dot_general(a_t, m_ref[...], (((0,), (0,)), ((), ())),
                            preferred_element_type=jnp.float32), 0.0)
    h2 = jnp.maximum(
        jnp.dot(h1, w2_ref[...], preferred_element_type=jnp.float32)
        + b2_ref[...], 0.0)
    out_ref[...] = (jnp.dot(h2, w3_ref[...], preferred_element_type=jnp.float32)
                    + b3_ref[...])


@jax.jit
def kernel(school_idx, grade_idx, goal_idx, subject_idx, method_idx,
           school_table, grade_table, goal_table, subject_table, method_table,
           W1, b1, W2, b2, W3, b3):
    grid = B // BLK
    idxs = [i.astype(jnp.int32).reshape(grid, 1, BLK)
            for i in (school_idx, grade_idx, goal_idx, subject_idx,
                      method_idx)]
    idx_spec = pl.BlockSpec((1, 1, BLK), lambda i: (i, 0, 0))
    full = lambda s: pl.BlockSpec(s, lambda i: tuple(0 for _ in s))
    out = pl.pallas_call(
        _body,
        grid=(grid,),
        in_specs=[idx_spec] * 5 + [
            full((VSIZES[0], EMB)), full((VSIZES[1], EMB)),
            full((VSIZES[2], EMB)), full((VSIZES[3], EMB)),
            full((VSIZES[4], EMB)),
            full((5 * EMB, 256)), full((1, 256)),
            full((256, 128)), full((1, 128)),
            full((128, 32)), full((1, 32)),
        ],
        out_specs=pl.BlockSpec((BLK, 32), lambda i: (i, 0)),
        out_shape=jax.ShapeDtypeStruct((B, 32), jnp.float32),
        scratch_shapes=[pltpu.VMEM((VPAD, 256), jnp.float32)],
        compiler_params=pltpu.CompilerParams(
            dimension_semantics=("arbitrary",)),
    )(*idxs, school_table, grade_table, goal_table, subject_table,
      method_table, W1, b1.reshape(1, 256), W2, b2.reshape(1, 128),
      W3, b3.reshape(1, 32))
    return out


# speedup vs baseline: 1.0620x; 1.0620x over previous
import jax
import jax.numpy as jnp
from jax.experimental import pallas as pl
from jax.experimental.pallas import tpu as pltpu

B = 16384
EMB = 32
VSIZES = (52, 14, 12, 14, 8)
PV = (56, 16, 16, 16, 8)
POFF = (0, 56, 72, 88, 104)
VPAD = 128
BLK = 4096


def _body(si_ref, gi_ref, oi_ref, ui_ref, mi_ref,
          st_ref, gt_ref, ot_ref, ut_ref, mt_ref,
          w1_ref, b1_ref, w2_ref, b2_ref, w3_ref, b3_ref,
          out_ref, m_ref):
    @pl.when(pl.program_id(0) == 0)
    def _fold():
        m_ref[...] = jnp.zeros((VPAD, 256), jnp.float32)
        for f, t_ref in enumerate((st_ref, gt_ref, ot_ref, ut_ref, mt_ref)):
            t = t_ref[...]
            if PV[f] > VSIZES[f]:
                t = jnp.concatenate(
                    [t, jnp.zeros((PV[f] - VSIZES[f], EMB), jnp.float32)], 0)
            w1f = w1_ref[f * EMB:(f + 1) * EMB, :]
            p = jnp.dot(t, w1f, preferred_element_type=jnp.float32)
            if f == 4:
                p = p + b1_ref[...]
            m_ref[POFF[f]:POFF[f] + PV[f], :] = p

    pieces = []
    for f, i_ref in enumerate((si_ref, gi_ref, oi_ref, ui_ref, mi_ref)):
        iota = jax.lax.broadcasted_iota(jnp.int32, (PV[f], BLK), 0)
        pieces.append((iota == i_ref[0]).astype(jnp.float32))
    a_t = jnp.concatenate(pieces, axis=0)

    h1 = jnp.maximum(
        jax.lax.dot_general(a_t, m_ref[0:sum(PV), :],
                            (((0,), (0,)), ((), ())),
                            preferred_element_type=jnp.float32), 0.0)
    h2 = jnp.maximum(
        jnp.dot(h1, w2_ref[...], preferred_element_type=jnp.float32)
        + b2_ref[...], 0.0)
    out_ref[...] = (jnp.dot(h2, w3_ref[...], preferred_element_type=jnp.float32)
                    + b3_ref[...])


@jax.jit
def kernel(school_idx, grade_idx, goal_idx, subject_idx, method_idx,
           school_table, grade_table, goal_table, subject_table, method_table,
           W1, b1, W2, b2, W3, b3):
    grid = B // BLK
    idxs = [i.astype(jnp.int32).reshape(grid, 1, BLK)
            for i in (school_idx, grade_idx, goal_idx, subject_idx,
                      method_idx)]
    idx_spec = pl.BlockSpec((1, 1, BLK), lambda i: (i, 0, 0))
    full = lambda s: pl.BlockSpec(s, lambda i: tuple(0 for _ in s))
    out = pl.pallas_call(
        _body,
        grid=(grid,),
        in_specs=[idx_spec] * 5 + [
            full((VSIZES[0], EMB)), full((VSIZES[1], EMB)),
            full((VSIZES[2], EMB)), full((VSIZES[3], EMB)),
            full((VSIZES[4], EMB)),
            full((5 * EMB, 256)), full((1, 256)),
            full((256, 128)), full((1, 128)),
            full((128, 32)), full((1, 32)),
        ],
        out_specs=pl.BlockSpec((BLK, 32), lambda i: (i, 0)),
        out_shape=jax.ShapeDtypeStruct((B, 32), jnp.float32),
        scratch_shapes=[pltpu.VMEM((VPAD, 256), jnp.float32)],
        compiler_params=pltpu.CompilerParams(
            dimension_semantics=("arbitrary",)),
    )(*idxs, school_table, grade_table, goal_table, subject_table,
      method_table, W1, b1.reshape(1, 256), W2, b2.reshape(1, 128),
      W3, b3.reshape(1, 32))
    return out


# speedup vs baseline: 1.0661x; 1.0038x over previous
import jax
import jax.numpy as jnp
from jax.experimental import pallas as pl
from jax.experimental.pallas import tpu as pltpu

B = 16384
EMB = 32
VSIZES = (52, 14, 12, 14, 8)
PV = (56, 16, 16, 16, 8)
POFF = (0, 56, 72, 88, 104)
VPAD = 128
BLK = 8192


def _body(si_ref, gi_ref, oi_ref, ui_ref, mi_ref,
          st_ref, gt_ref, ot_ref, ut_ref, mt_ref,
          w1_ref, b1_ref, w2_ref, b2_ref, w3_ref, b3_ref,
          out_ref, m_ref):
    @pl.when(pl.program_id(0) == 0)
    def _fold():
        m_ref[...] = jnp.zeros((VPAD, 256), jnp.float32)
        for f, t_ref in enumerate((st_ref, gt_ref, ot_ref, ut_ref, mt_ref)):
            t = t_ref[...]
            if PV[f] > VSIZES[f]:
                t = jnp.concatenate(
                    [t, jnp.zeros((PV[f] - VSIZES[f], EMB), jnp.float32)], 0)
            w1f = w1_ref[f * EMB:(f + 1) * EMB, :]
            p = jnp.dot(t, w1f, preferred_element_type=jnp.float32)
            if f == 4:
                p = p + b1_ref[...]
            m_ref[POFF[f]:POFF[f] + PV[f], :] = p

    pieces = []
    for f, i_ref in enumerate((si_ref, gi_ref, oi_ref, ui_ref, mi_ref)):
        iota = jax.lax.broadcasted_iota(jnp.int32, (PV[f], BLK), 0)
        pieces.append((iota == i_ref[0]).astype(jnp.float32))
    a_t = jnp.concatenate(pieces, axis=0)

    h1 = jnp.maximum(
        jax.lax.dot_general(a_t, m_ref[0:sum(PV), :],
                            (((0,), (0,)), ((), ())),
                            preferred_element_type=jnp.float32), 0.0)
    h2 = jnp.maximum(
        jnp.dot(h1, w2_ref[...], preferred_element_type=jnp.float32)
        + b2_ref[...], 0.0)
    out_ref[...] = (jnp.dot(h2, w3_ref[...], preferred_element_type=jnp.float32)
                    + b3_ref[...])


@jax.jit
def kernel(school_idx, grade_idx, goal_idx, subject_idx, method_idx,
           school_table, grade_table, goal_table, subject_table, method_table,
           W1, b1, W2, b2, W3, b3):
    grid = B // BLK
    idxs = [i.astype(jnp.int32).reshape(grid, 1, BLK)
            for i in (school_idx, grade_idx, goal_idx, subject_idx,
                      method_idx)]
    idx_spec = pl.BlockSpec((1, 1, BLK), lambda i: (i, 0, 0))
    full = lambda s: pl.BlockSpec(s, lambda i: tuple(0 for _ in s))
    out = pl.pallas_call(
        _body,
        grid=(grid,),
        in_specs=[idx_spec] * 5 + [
            full((VSIZES[0], EMB)), full((VSIZES[1], EMB)),
            full((VSIZES[2], EMB)), full((VSIZES[3], EMB)),
            full((VSIZES[4], EMB)),
            full((5 * EMB, 256)), full((1, 256)),
            full((256, 128)), full((1, 128)),
            full((128, 32)), full((1, 32)),
        ],
        out_specs=pl.BlockSpec((BLK, 32), lambda i: (i, 0)),
        out_shape=jax.ShapeDtypeStruct((B, 32), jnp.float32),
        scratch_shapes=[pltpu.VMEM((VPAD, 256), jnp.float32)],
        compiler_params=pltpu.CompilerParams(
            dimension_semantics=("arbitrary",)),
    )(*idxs, school_table, grade_table, goal_table, subject_table,
      method_table, W1, b1.reshape(1, 256), W2, b2.reshape(1, 128),
      W3, b3.reshape(1, 32))
    return out
